# Initial kernel scaffold; baseline (speedup 1.0000x reference)
import jax
import jax.numpy as jnp
from jax.experimental import pallas as pl


def kernel(x, pos, edge_index, node_subnode_index, subgraph_edge_index, subnode_node_index, batch, params):
    raise NotImplementedError("write your pallas kernel here")


# speedup vs baseline: 2.7885x; 2.7885x over previous
import functools

import jax
import jax.numpy as jnp
from jax import lax
from jax.experimental import pallas as pl
from jax.experimental.pallas import tpu as pltpu
from jax.experimental.pallas import tpu_sc as plsc

NNODE = 10000
HD = 128
NG = 64
NP = 10240
NE = 160000
EP = 163840
NC = 2
NS = 16
NW = NC * NS
EW = EP // NW
CHUNK = 128
NCHUNK = EW // CHUNK
DCH = 512
RT = NP // NS
BLKE = 512
BLKN = 512


@functools.lru_cache(maxsize=None)
def _sc_kernels():
    mesh = plsc.VectorSubcoreMesh(core_axis_name="c", subcore_axis_name="s")

    @functools.partial(
        pl.kernel,
        compiler_params=pltpu.CompilerParams(needs_layout_passes=False),
        out_type=jax.ShapeDtypeStruct((EP,), jnp.float32),
        mesh=mesh,
        scratch_types=[
            pltpu.VMEM((NP * 4,), jnp.float32),
            pltpu.VMEM((DCH,), jnp.int32),
            pltpu.VMEM((DCH,), jnp.int32),
            pltpu.VMEM((DCH,), jnp.float32),
        ],
    )
    def sc_dist(posf, srcs, dsts, out, ptab, sb, db, dout):
        wid = lax.axis_index("s") * NC + lax.axis_index("c")
        pltpu.sync_copy(posf, ptab)
        half = jnp.full((16,), 0.5, jnp.float32)
        thalf = jnp.full((16,), 1.5, jnp.float32)
        magic = jnp.full((16,), 0x5F3759DF, jnp.int32)
        tiny = jnp.full((16,), 1e-30, jnp.float32)

        def chunk(ci, carry):
            base = wid * EW + ci * DCH
            pltpu.sync_copy(srcs.at[pl.ds(base, DCH)], sb)
            pltpu.sync_copy(dsts.at[pl.ds(base, DCH)], db)
            for i in range(DCH // 16):
                sl = pl.ds(i * 16, 16)
                a = sb[sl] * 4
                b = db[sl] * 4
                dx = (plsc.load_gather(ptab, [b])
                      - plsc.load_gather(ptab, [a]))
                dy = (plsc.load_gather(ptab, [b + 1])
                      - plsc.load_gather(ptab, [a + 1]))
                dz = (plsc.load_gather(ptab, [b + 2])
                      - plsc.load_gather(ptab, [a + 2]))
                d2 = dx * dx + dy * dy + dz * dz
                d2c = jnp.maximum(d2, tiny)
                bits = plsc.bitcast(d2c, jnp.int32)
                y = plsc.bitcast(magic - lax.shift_right_logical(bits, 1),
                                 jnp.float32)
                hs = half * d2c
                y = y * (thalf - hs * y * y)
                y = y * (thalf - hs * y * y)
                y = y * (thalf - hs * y * y)
                dout[sl] = d2 * y
            pltpu.sync_copy(dout, out.at[pl.ds(base, DCH)])
            return carry

        lax.fori_loop(0, EW // DCH, chunk, 0)

    @functools.partial(
        pl.kernel,
        compiler_params=pltpu.CompilerParams(needs_layout_passes=False),
        out_type=jax.ShapeDtypeStruct((EP, HD), jnp.float32),
        mesh=mesh,
        scratch_types=[
            pltpu.VMEM((CHUNK,), jnp.int32),
            pltpu.VMEM((CHUNK,), jnp.int32),
            pltpu.VMEM((CHUNK, HD), jnp.float32),
            pltpu.VMEM((CHUNK, HD), jnp.float32),
            pltpu.SemaphoreType.DMA,
            pltpu.SemaphoreType.DMA,
        ],
    )
    def sc_gather(tabd, tabs, idxd, idxs, out, idxb_d, idxb_s, buf_d, buf_s,
                  sem_d, sem_s):
        wid = lax.axis_index("s") * NC + lax.axis_index("c")

        def chunk(ci, carry):
            base = wid * EW + ci * CHUNK
            pltpu.sync_copy(idxd.at[pl.ds(base, CHUNK)], idxb_d)
            pltpu.sync_copy(idxs.at[pl.ds(base, CHUNK)], idxb_s)
            cp_d = pltpu.async_copy(tabd.at[idxb_d], buf_d, sem_d)
            cp_s = pltpu.async_copy(tabs.at[idxb_s], buf_s, sem_s)
            cp_d.wait()
            cp_s.wait()

            def addrow(r, c2):
                for c in range(HD // 16):
                    sl = pl.ds(c * 16, 16)
                    buf_d[r, sl] = buf_d[r, sl] + buf_s[r, sl]
                return c2

            lax.fori_loop(0, CHUNK, addrow, 0)
            pltpu.sync_copy(buf_d, out.at[pl.ds(base, CHUNK)])
            return carry

        lax.fori_loop(0, NCHUNK, chunk, 0)

    @functools.partial(
        pl.kernel,
        compiler_params=pltpu.CompilerParams(needs_layout_passes=False),
        out_type=jax.ShapeDtypeStruct((NC * NP, HD), jnp.float32),
        mesh=mesh,
        scratch_types=[
            pltpu.VMEM((CHUNK,), jnp.int32),
            pltpu.VMEM((CHUNK, HD), jnp.float32),
            pltpu.VMEM_SHARED((NP, HD), jnp.float32),
        ],
    )
    def sc_scatter(m2, sidx, out, idxb, rows, acc):
        cid = lax.axis_index("c")
        sid = lax.axis_index("s")
        wid = sid * NC + cid

        def zrow(r, carry):
            for c in range(HD // 16):
                rows[r, pl.ds(c * 16, 16)] = jnp.zeros((16,), jnp.float32)
            return carry

        lax.fori_loop(0, CHUNK, zrow, 0)
        for k in range(RT // CHUNK):
            pltpu.sync_copy(rows, acc.at[pl.ds(sid * RT + k * CHUNK, CHUNK)])
        plsc.subcore_barrier()

        def chunk(ci, carry):
            base = wid * EW + ci * CHUNK
            pltpu.sync_copy(sidx.at[pl.ds(base, CHUNK)], idxb)
            pltpu.sync_copy(m2.at[pl.ds(base, CHUNK)], rows)
            pltpu.sync_copy(rows, acc.at[idxb], add=True)
            return carry

        lax.fori_loop(0, NCHUNK, chunk, 0)
        plsc.subcore_barrier()

        for k in range(RT // CHUNK):
            r0 = sid * RT + k * CHUNK
            pltpu.sync_copy(acc.at[pl.ds(r0, CHUNK)], rows)
            pltpu.sync_copy(rows, out.at[pl.ds(cid * NP + r0, CHUNK)])

    return sc_dist, sc_gather, sc_scatter


def _sc_dist(posf, srcs, dsts):
    return _sc_kernels()[0](posf, srcs, dsts)


def _sc_gather(tabd, tabs, idxd, idxs):
    return _sc_kernels()[1](tabd, tabs, idxd, idxs)


def _sc_scatter(m2, sidx):
    return _sc_kernels()[2](m2, sidx)


def _ln_swish(x, g, b):
    mu = jnp.mean(x, axis=-1, keepdims=True)
    xc = x - mu
    v = jnp.mean(xc * xc, axis=-1, keepdims=True)
    y = xc * lax.rsqrt(v + 1e-5) * g[None, :] + b[None, :]
    return y * jax.nn.sigmoid(y)


def _edge_body(g_ref, d_ref, vp_ref, w2_ref, out_ref):
    G = g_ref[...]
    vp = vp_ref[...]
    d = d_ref[...]
    pre = G + d * vp[0][None, :] + vp[1][None, :]
    m = _ln_swish(pre, vp[2], vp[3])
    pre2 = jnp.dot(m, w2_ref[...], preferred_element_type=jnp.float32)
    out_ref[...] = _ln_swish(pre2 + vp[4][None, :], vp[5], vp[6])


def _edge_call(G, dcol, vp, w2):
    return pl.pallas_call(
        _edge_body,
        grid=(EP // BLKE,),
        in_specs=[
            pl.BlockSpec((BLKE, HD), lambda i: (i, 0)),
            pl.BlockSpec((BLKE, 1), lambda i: (i, 0)),
            pl.BlockSpec((8, HD), lambda i: (0, 0)),
            pl.BlockSpec((HD, HD), lambda i: (0, 0)),
        ],
        out_specs=pl.BlockSpec((BLKE, HD), lambda i: (i, 0)),
        out_shape=jax.ShapeDtypeStruct((EP, HD), jnp.float32),
    )(G, dcol, vp, w2)


def _node_body(produce, has_res, *refs):
    h_ref, parts_ref, vp_ref, u1a_ref, u1b_ref, u2_ref = refs[:6]
    idx = 6
    if has_res:
        h0_ref = refs[idx]
        idx += 1
    if produce:
        w1t_ref, w1s_ref = refs[idx:idx + 2]
        idx += 2
    hn_ref = refs[idx]
    idx += 1
    if produce:
        tabd_ref, tabs_ref = refs[idx:idx + 2]

    hv = h_ref[...]
    agg = parts_ref[0] + parts_ref[1]
    vp = vp_ref[...]
    pre = (jnp.dot(hv, u1a_ref[...], preferred_element_type=jnp.float32)
           + jnp.dot(agg, u1b_ref[...], preferred_element_type=jnp.float32)
           + vp[0][None, :])
    u = _ln_swish(pre, vp[1], vp[2])
    pre2 = jnp.dot(u, u2_ref[...], preferred_element_type=jnp.float32)
    u2v = _ln_swish(pre2 + vp[3][None, :], vp[4], vp[5])
    if has_res:
        u2v = u2v + h0_ref[...]
    hn_ref[...] = u2v
    if produce:
        tabd_ref[...] = jnp.dot(u2v, w1t_ref[...],
                                preferred_element_type=jnp.float32)
        tabs_ref[...] = jnp.dot(u2v, w1s_ref[...],
                                preferred_element_type=jnp.float32)


def _node_call(h, parts, vp, u1a, u1b, u2m, h0=None, w1t=None, w1s=None):
    produce = w1t is not None
    has_res = h0 is not None
    mat = pl.BlockSpec((HD, HD), lambda i: (0, 0))
    blk = pl.BlockSpec((BLKN, HD), lambda i: (i, 0))
    in_specs = [
        blk,
        pl.BlockSpec((2, BLKN, HD), lambda i: (0, i, 0)),
        pl.BlockSpec((8, HD), lambda i: (0, 0)),
        mat, mat, mat,
    ]
    args = [h, parts, vp, u1a, u1b, u2m]
    if has_res:
        in_specs.append(blk)
        args.append(h0)
    if produce:
        in_specs += [mat, mat]
        args += [w1t, w1s]
    out_specs = [blk]
    out_shape = [jax.ShapeDtypeStruct((NP, HD), jnp.float32)]
    if produce:
        out_specs += [blk, blk]
        out_shape += [jax.ShapeDtypeStruct((NP, HD), jnp.float32)] * 2
    res = pl.pallas_call(
        functools.partial(_node_body, produce, has_res),
        grid=(NP // BLKN,),
        in_specs=in_specs,
        out_specs=out_specs,
        out_shape=out_shape,
    )(*args)
    if produce:
        return res[0], res[1], res[2]
    return res[0], None, None


def _embed_body(x_ref, ew_ref, eb_ref, w1t_ref, w1s_ref,
                h_ref, tabd_ref, tabs_ref):
    hv = (jnp.dot(x_ref[...], ew_ref[...], preferred_element_type=jnp.float32)
          + eb_ref[0][None, :])
    h_ref[...] = hv
    tabd_ref[...] = jnp.dot(hv, w1t_ref[...],
                            preferred_element_type=jnp.float32)
    tabs_ref[...] = jnp.dot(hv, w1s_ref[...],
                            preferred_element_type=jnp.float32)


def _embed_call(xp, ew, eb, w1t, w1s):
    mat = pl.BlockSpec((HD, HD), lambda i: (0, 0))
    blk = pl.BlockSpec((BLKN, HD), lambda i: (i, 0))
    return pl.pallas_call(
        _embed_body,
        grid=(NP // BLKN,),
        in_specs=[blk, mat, pl.BlockSpec((1, HD), lambda i: (0, 0)),
                  mat, mat],
        out_specs=[blk, blk, blk],
        out_shape=[jax.ShapeDtypeStruct((NP, HD), jnp.float32)] * 3,
    )(xp, ew, eb, w1t, w1s)


def _pool_body(h_ref, b_ref, pw1_ref, pw2_ref, pb_ref, out_ref, acc_ref):
    i = pl.program_id(0)

    @pl.when(i == 0)
    def _():
        acc_ref[...] = jnp.zeros_like(acc_ref)

    b = b_ref[...]
    iota = lax.broadcasted_iota(jnp.int32, (BLKN, NG), 1).astype(jnp.float32)
    oh = (b == iota).astype(jnp.float32)
    acc_ref[...] += lax.dot_general(
        oh, h_ref[...], (((0,), (0,)), ((), ())),
        preferred_element_type=jnp.float32)

    @pl.when(i == pl.num_programs(0) - 1)
    def _():
        pooled = acc_ref[...]
        hid = jnp.maximum(
            jnp.dot(pooled, pw1_ref[...], preferred_element_type=jnp.float32)
            + pb_ref[0][None, :], 0.0)
        out_ref[...] = (jnp.dot(hid, pw2_ref[...],
                                preferred_element_type=jnp.float32)
                        + pb_ref[1][None, :])


def _pool_call(h, bcol, pw1, pw2p, pb):
    return pl.pallas_call(
        _pool_body,
        grid=(NP // BLKN,),
        in_specs=[
            pl.BlockSpec((BLKN, HD), lambda i: (i, 0)),
            pl.BlockSpec((BLKN, 1), lambda i: (i, 0)),
            pl.BlockSpec((HD, HD), lambda i: (0, 0)),
            pl.BlockSpec((HD, HD), lambda i: (0, 0)),
            pl.BlockSpec((2, HD), lambda i: (0, 0)),
        ],
        out_specs=pl.BlockSpec((NG, HD), lambda i: (0, 0)),
        out_shape=jax.ShapeDtypeStruct((NG, HD), jnp.float32),
        scratch_shapes=[pltpu.VMEM((NG, HD), jnp.float32)],
    )(h, bcol, pw1, pw2p, pb)


def _edge_vec(p):
    return jnp.stack([
        p["msg_W1"][2 * HD], p["msg_b1"], p["msg_g1"], p["msg_be1"],
        p["msg_b2"], p["msg_g2"], p["msg_be2"],
        jnp.zeros((HD,), jnp.float32),
    ])


def _node_vec(p):
    z = jnp.zeros((HD,), jnp.float32)
    return jnp.stack([
        p["upd_b1"], p["upd_g1"], p["upd_be1"],
        p["upd_b2"], p["upd_g2"], p["upd_be2"], z, z,
    ])


def kernel(x, pos, edge_index, node_subnode_index, subgraph_edge_index,
           subnode_node_index, batch, params):
    f32 = jnp.float32
    xp = jnp.zeros((NP, HD), f32).at[:NNODE].set(x)
    posf = jnp.zeros((NP, 4), f32).at[:NNODE, :3].set(pos).reshape(-1)
    bcol = jnp.full((NP, 1), float(NG), f32).at[:NNODE, 0].set(
        batch.astype(f32))

    pad_g = (jnp.arange(EP - NE, dtype=jnp.int32) % NNODE)
    pad_s = NNODE + (jnp.arange(EP - NE, dtype=jnp.int32) % (NP - NNODE))
    edge_sets = [edge_index, node_subnode_index, subgraph_edge_index,
                 subnode_node_index]
    gsrc, gdst, sdst, dcol = [], [], [], []
    for es in edge_sets:
        src = es[0].astype(jnp.int32)
        dst = es[1].astype(jnp.int32)
        gsrc.append(jnp.concatenate([src, pad_g]))
        gdst.append(jnp.concatenate([dst, pad_g]))
        sdst.append(jnp.concatenate([dst, pad_s]))
        dcol.append(_sc_dist(posf, gsrc[-1], gdst[-1]).reshape(EP, 1))

    lp = [params["layers"][l][t] for l in range(2) for t in range(4)]
    w1t = [p["msg_W1"][:HD] for p in lp]
    w1s = [p["msg_W1"][HD:2 * HD] for p in lp]

    h, tabd, tabs = _embed_call(
        xp, params["emb_W"], params["emb_b"][None, :], w1t[0], w1s[0])
    h0 = h
    for k in range(8):
        t = k % 4
        p = lp[k]
        G = _sc_gather(tabd, tabs, gdst[t], gsrc[t])
        m2 = _edge_call(G, dcol[t], _edge_vec(p), p["msg_W2"])
        parts = _sc_scatter(m2, sdst[t]).reshape(NC, NP, HD)
        res = h0 if t == 3 else None
        nxt_t = w1t[k + 1] if k < 7 else None
        nxt_s = w1s[k + 1] if k < 7 else None
        h, tabd, tabs = _node_call(
            h, parts, _node_vec(p), p["upd_W1"][:HD], p["upd_W1"][HD:],
            p["upd_W2"], h0=res, w1t=nxt_t, w1s=nxt_s)
        if t == 3:
            h0 = h

    pw2p = jnp.zeros((HD, HD), f32).at[:, :1].set(params["pred_W2"])
    pb = jnp.stack([params["pred_b1"],
                    jnp.zeros((HD,), f32).at[0].set(params["pred_b2"][0])])
    pooled = _pool_call(h, bcol, params["pred_W1"], pw2p, pb)
    return pooled[:, :1]
